# baseline (speedup 1.0000x reference)
import functools

import jax
import jax.numpy as jnp
from jax import lax
from jax.experimental import pallas as pl
from jax.experimental.pallas import tpu as pltpu
from jax.experimental.pallas import tpu_sc as plsc

NC = 2
NS = 16
L = 16
NW = NC * NS

B = 16384
D = 32
BPW = B // NW
NBUF = 4
NG = BPW // L

_mesh = plsc.VectorSubcoreMesh(core_axis_name="c", subcore_axis_name="s")


@functools.partial(
    pl.kernel,
    out_type=jax.ShapeDtypeStruct((B,), jnp.float32),
    mesh=_mesh,
    scratch_types=[
        pltpu.VMEM((BPW,), jnp.int32),
        pltpu.VMEM((BPW,), jnp.int32),
        pltpu.VMEM_SHARED((NS, NBUF, D, 128), jnp.float32),
        pltpu.VMEM_SHARED((NS, NBUF, D, 128), jnp.float32),
        pltpu.VMEM((NBUF, D, 128), jnp.float32),
        pltpu.VMEM((NBUF, D, 128), jnp.float32),
        pltpu.VMEM((BPW * D,), jnp.float32),
        pltpu.VMEM((BPW * D,), jnp.float32),
        pltpu.VMEM((BPW,), jnp.float32),
        pltpu.SemaphoreType.DMA,
        pltpu.SemaphoreType.DMA,
        pltpu.SemaphoreType.DMA,
    ],
    compiler_params=pltpu.CompilerParams(
        needs_layout_passes=False, use_tc_tiling_on_sc=True
    ),
)
def _mf_kernel(uid_hbm, iid_hbm, ut_hbm, it_hbm, out_hbm,
               uids_v, iids_v, ublk, iblk, ustg, istg, urows, irows, out_v,
               sem_u, sem_i, sem_x):
    wid = lax.axis_index("s") * NC + lax.axis_index("c")
    sid = lax.axis_index("s")
    base = wid * BPW

    pltpu.sync_copy(uid_hbm.at[pl.ds(base, BPW)], uids_v)
    pltpu.sync_copy(iid_hbm.at[pl.ds(base, BPW)], iids_v)

    lane = lax.iota(jnp.int32, L)

    def group(g, carry):
        off = g * L
        ruv = uids_v[pl.ds(off, L)]
        riv = iids_v[pl.ds(off, L)]

        for half in range(L // NBUF):
            for i in range(NBUF):
                r_u = ruv[half * NBUF + i]
                r_i = riv[half * NBUF + i]
                rb_u = pl.multiple_of((r_u >> 7) * 128, 128)
                rb_i = pl.multiple_of((r_i >> 7) * 128, 128)
                pltpu.async_copy(
                    ut_hbm.at[:, pl.ds(rb_u, 128)], ublk.at[sid, i], sem_u
                )
                pltpu.async_copy(
                    it_hbm.at[:, pl.ds(rb_i, 128)], iblk.at[sid, i], sem_i
                )
            for i in range(NBUF):
                pltpu.make_async_copy(
                    ut_hbm.at[:, pl.ds(0, 128)], ublk.at[0, 0], sem_u
                ).wait()
                pltpu.make_async_copy(
                    it_hbm.at[:, pl.ds(0, 128)], iblk.at[0, 0], sem_i
                ).wait()
            for i in range(NBUF):
                row = half * NBUF + i
                cu = ruv[row] & 127
                ci = riv[row] & 127
                pltpu.async_copy(
                    ublk.at[sid, i, :, pl.ds(cu, 1)],
                    ustg.at[i, :, pl.ds(0, 1)],
                    sem_x,
                )
                pltpu.async_copy(
                    iblk.at[sid, i, :, pl.ds(ci, 1)],
                    istg.at[i, :, pl.ds(0, 1)],
                    sem_x,
                )
            for i in range(NBUF):
                pltpu.make_async_copy(
                    ublk.at[0, 0, :, pl.ds(0, 1)],
                    ustg.at[0, :, pl.ds(0, 1)],
                    sem_x,
                ).wait()
                pltpu.make_async_copy(
                    ublk.at[0, 0, :, pl.ds(0, 1)],
                    istg.at[0, :, pl.ds(0, 1)],
                    sem_x,
                ).wait()
            zero16 = jnp.zeros((L,), jnp.int32)
            d_lo = lane
            d_hi = lane + L
            for i in range(NBUF):
                row = half * NBUF + i
                w = (off + row) * D
                slot = jnp.full((L,), i, jnp.int32)
                urows[pl.ds(w, L)] = plsc.load_gather(ustg, [slot, d_lo, zero16])
                urows[pl.ds(w + L, L)] = plsc.load_gather(ustg, [slot, d_hi, zero16])
                irows[pl.ds(w, L)] = plsc.load_gather(istg, [slot, d_lo, zero16])
                irows[pl.ds(w + L, L)] = plsc.load_gather(istg, [slot, d_hi, zero16])

        acc = jnp.zeros((L,), jnp.float32)
        word0 = (off + lane) * D
        for d in range(D):
            u = plsc.load_gather(urows, [word0 + d])
            v = plsc.load_gather(irows, [word0 + d])
            acc = acc + u * v
        out_v[pl.ds(off, L)] = acc
        return carry

    lax.fori_loop(0, NG, group, 0)

    pltpu.sync_copy(out_v, out_hbm.at[pl.ds(base, BPW)])


def kernel(user_ids, item_ids, user_table, item_table):
    uid = user_ids.astype(jnp.int32)
    iid = item_ids.astype(jnp.int32)
    out = _mf_kernel(uid, iid, user_table.T, item_table.T)
    return out.reshape(B, 1)


# speedup vs baseline: 1.3900x; 1.3900x over previous
import functools

import jax
import jax.numpy as jnp
from jax import lax
from jax.experimental import pallas as pl
from jax.experimental.pallas import tpu as pltpu
from jax.experimental.pallas import tpu_sc as plsc

NC = 2
NS = 16
L = 16
NW = NC * NS

B = 16384
D = 32
BPW = B // NW
NBUF = 8
NG = BPW // L

_mesh = plsc.VectorSubcoreMesh(core_axis_name="c", subcore_axis_name="s")


@functools.partial(
    pl.kernel,
    out_type=jax.ShapeDtypeStruct((B,), jnp.float32),
    mesh=_mesh,
    scratch_types=[
        pltpu.VMEM((BPW,), jnp.int32),
        pltpu.VMEM((BPW,), jnp.int32),
        pltpu.VMEM((NBUF, D, 128), jnp.float32),
        pltpu.VMEM((NBUF, D, 128), jnp.float32),
        pltpu.VMEM((L, 128), jnp.float32),
        pltpu.VMEM((L, 128), jnp.float32),
        pltpu.VMEM((BPW,), jnp.float32),
        pltpu.SemaphoreType.DMA,
        pltpu.SemaphoreType.DMA,
    ],
    compiler_params=pltpu.CompilerParams(
        needs_layout_passes=False, use_tc_tiling_on_sc=True
    ),
)
def _mf_kernel(uid_hbm, iid_hbm, ut_hbm, it_hbm, out_hbm,
               uids_v, iids_v, ublk, iblk, ustage, istage, out_v,
               sem_u, sem_i):
    wid = lax.axis_index("s") * NC + lax.axis_index("c")
    base = wid * BPW

    pltpu.sync_copy(uid_hbm.at[pl.ds(base, BPW)], uids_v)
    pltpu.sync_copy(iid_hbm.at[pl.ds(base, BPW)], iids_v)

    lane = lax.iota(jnp.int32, L)
    d_lo = lane
    d_hi = lane + L

    def group(g, carry):
        off = g * L
        ruv = uids_v[pl.ds(off, L)]
        riv = iids_v[pl.ds(off, L)]

        for half in range(2):
            for i in range(NBUF):
                r_u = ruv[half * NBUF + i]
                r_i = riv[half * NBUF + i]
                rb_u = pl.multiple_of((r_u >> 7) * 128, 128)
                rb_i = pl.multiple_of((r_i >> 7) * 128, 128)
                pltpu.async_copy(
                    ut_hbm.at[:, pl.ds(rb_u, 128)], ublk.at[i], sem_u
                )
                pltpu.async_copy(
                    it_hbm.at[:, pl.ds(rb_i, 128)], iblk.at[i], sem_i
                )
            for i in range(NBUF):
                pltpu.make_async_copy(
                    ut_hbm.at[:, pl.ds(0, 128)], ublk.at[0], sem_u
                ).wait()
                pltpu.make_async_copy(
                    it_hbm.at[:, pl.ds(0, 128)], iblk.at[0], sem_i
                ).wait()
            for i in range(NBUF):
                row = half * NBUF + i
                cu = jnp.full((L,), ruv[row] & 127, jnp.int32)
                ci = jnp.full((L,), riv[row] & 127, jnp.int32)
                slot = jnp.full((L,), i, jnp.int32)
                ustage[row, pl.ds(0, L)] = plsc.load_gather(ublk, [slot, d_lo, cu])
                ustage[row, pl.ds(L, L)] = plsc.load_gather(ublk, [slot, d_hi, cu])
                istage[row, pl.ds(0, L)] = plsc.load_gather(iblk, [slot, d_lo, ci])
                istage[row, pl.ds(L, L)] = plsc.load_gather(iblk, [slot, d_hi, ci])

        acc = jnp.zeros((L,), jnp.float32)
        for d in range(D):
            dv = jnp.full((L,), d, jnp.int32)
            u = plsc.load_gather(ustage, [lane, dv])
            v = plsc.load_gather(istage, [lane, dv])
            acc = acc + u * v
        out_v[pl.ds(off, L)] = acc
        return carry

    lax.fori_loop(0, NG, group, 0)

    pltpu.sync_copy(out_v, out_hbm.at[pl.ds(base, BPW)])


def kernel(user_ids, item_ids, user_table, item_table):
    uid = user_ids.astype(jnp.int32)
    iid = item_ids.astype(jnp.int32)
    out = _mf_kernel(uid, iid, user_table.T, item_table.T)
    return out.reshape(B, 1)
